# baseline (speedup 1.0000x reference)
import functools

import jax
import jax.numpy as jnp
from jax import lax
from jax.experimental import pallas as pl
from jax.experimental.pallas import tpu as pltpu
from jax.experimental.pallas import tpu_sc as plsc

N_NODES = 10000
D = 128
NC = 2
NS = 16
NW = NC * NS
CHUNK = 120
NBUF = 3
NIDX = 6
NCH0 = 168
NCH1 = 0
ROWS_PER_TILE = 632
NR = NS * ROWS_PER_TILE
DUMMY_ROW = N_NODES


def _sc_agg_body(with_deg, feat, src, dst, zrows, *refs):
    if with_deg:
        (agg_out, deg_out, idx_s, idx_d, ones_v, zdeg_v, agg_sp,
         deg_sp) = refs[:8]
        rest = refs[8:]
    else:
        (agg_out, idx_s, idx_d, agg_sp) = refs[:4]
        rest = refs[4:]
    rows = rest[:NBUF]
    gsems = rest[NBUF:2 * NBUF]
    isems = rest[2 * NBUF:2 * NBUF + NIDX]

    cid = lax.axis_index("c")
    sid = lax.axis_index("s")
    n_chunks = jnp.where(cid == 0, NCH0, NCH1)
    row0 = jnp.where(cid == 0, sid * NCH0, NS * NCH0 + sid * NCH1)

    def fetch_idx(slot, k, sem):
        base = (row0 + k) * CHUNK
        pltpu.async_copy(src.at[pl.ds(base, CHUNK)], idx_s.at[slot], sem)
        pltpu.async_copy(dst.at[pl.ds(base, CHUNK)], idx_d.at[slot], sem)

    def wait_idx(slot, sem):
        pltpu.make_async_copy(src.at[pl.ds(0, CHUNK)], idx_s.at[slot],
                              sem).wait()
        pltpu.make_async_copy(dst.at[pl.ds(0, CHUNK)], idx_d.at[slot],
                              sem).wait()

    for r in range(NIDX):
        @pl.when(r < n_chunks)
        def _prime_fetch(r=r):
            fetch_idx(r, r, isems[r])

    base_r = sid * ROWS_PER_TILE
    pltpu.sync_copy(zrows, agg_sp.at[pl.ds(base_r, ROWS_PER_TILE)])
    if with_deg:
        for j in range(8):
            ones_v[pl.ds(j * 16, 16)] = jnp.ones((16,), jnp.float32)
        for j in range(640 // 16):
            zdeg_v[pl.ds(j * 16, 16)] = jnp.zeros((16,), jnp.float32)
        pltpu.sync_copy(zdeg_v.at[pl.ds(0, ROWS_PER_TILE)],
                        deg_sp.at[pl.ds(base_r, ROWS_PER_TILE)])
    plsc.subcore_barrier()

    for b in range(NBUF):
        @pl.when(b < n_chunks)
        def _prime_gather(b=b):
            wait_idx(b, isems[b])
            pltpu.async_copy(feat.at[idx_s.at[b]], rows[b], gsems[b])

    def round_(g, carry):
        for b in range(NIDX):
            k = g + b

            @pl.when(k < n_chunks)
            def _consume(b=b, k=k, rb=b % NBUF):
                pltpu.make_async_copy(feat.at[idx_s.at[b]], rows[rb],
                                      gsems[rb]).wait()
                pltpu.sync_copy(rows[rb], agg_sp.at[idx_d.at[b]], add=True)
                if with_deg:
                    pltpu.sync_copy(ones_v.at[pl.ds(0, CHUNK)],
                                    deg_sp.at[idx_d.at[b]], add=True)

            @pl.when(k + NIDX < n_chunks)
            def _refetch(b=b, k=k):
                fetch_idx(b, k + NIDX, isems[b])

            @pl.when(k + NBUF < n_chunks)
            def _issue_next(b=b, k=k, rb=b % NBUF):
                nslot = (b + NBUF) % NIDX
                wait_idx(nslot, isems[nslot])
                pltpu.async_copy(feat.at[idx_s.at[nslot]], rows[rb],
                                 gsems[rb])
        return carry

    max_nch = max(NCH0, NCH1)
    lax.fori_loop(0, max_nch // NIDX, lambda i, c: round_(i * NIDX, c), 0)
    plsc.subcore_barrier()

    pltpu.sync_copy(agg_sp.at[pl.ds(base_r, ROWS_PER_TILE)],
                    agg_out.at[cid, pl.ds(base_r, ROWS_PER_TILE)])
    if with_deg:
        pltpu.sync_copy(deg_sp.at[pl.ds(base_r, ROWS_PER_TILE)],
                        zdeg_v.at[pl.ds(0, ROWS_PER_TILE)])
        pltpu.sync_copy(zdeg_v.at[pl.ds(0, ROWS_PER_TILE)],
                        deg_out.at[pl.ds(cid * NR + base_r, ROWS_PER_TILE)])


def _make_sc_agg(with_deg):
    mesh = plsc.VectorSubcoreMesh(core_axis_name="c", subcore_axis_name="s")
    out_type = [jax.ShapeDtypeStruct((NC, NR, D), jnp.float32)]
    scratch = [
        pltpu.VMEM((NIDX, CHUNK), jnp.int32),
        pltpu.VMEM((NIDX, CHUNK), jnp.int32),
    ]
    if with_deg:
        out_type.append(jax.ShapeDtypeStruct((NC * NR,), jnp.float32))
        scratch.append(pltpu.VMEM((128,), jnp.float32))
        scratch.append(pltpu.VMEM((640,), jnp.float32))
        scratch.append(pltpu.VMEM_SHARED((NR, D), jnp.float32))
        scratch.append(pltpu.VMEM_SHARED((NR,), jnp.float32))
    else:
        scratch.append(pltpu.VMEM_SHARED((NR, D), jnp.float32))
    scratch += [pltpu.VMEM((CHUNK, D), jnp.float32) for _ in range(NBUF)]
    scratch += [pltpu.SemaphoreType.DMA for _ in range(NBUF + NIDX)]
    return pl.kernel(
        functools.partial(_sc_agg_body, with_deg),
        out_type=out_type if with_deg else out_type[0],
        mesh=mesh,
        scratch_types=scratch,
    )


def _tc_layer1(a_ref, d_ref, x_ref, wl_ref, wr_ref, bl_ref, o_ref):
    agg = a_ref[0] + a_ref[1]
    deg = jnp.maximum(d_ref[0] + d_ref[1], 1.0)
    mean = agg * (1.0 / deg)
    h = (jnp.dot(mean, wl_ref[...], preferred_element_type=jnp.float32)
         + bl_ref[...]
         + jnp.dot(x_ref[...], wr_ref[...], preferred_element_type=jnp.float32))
    o_ref[...] = jnp.maximum(h, 0.0)


def _tc_layer2(a_ref, d_ref, x_ref, wl_ref, wr_ref, bl_ref, wo_ref, bo_ref,
               o_ref):
    agg = a_ref[0] + a_ref[1]
    deg = jnp.maximum(d_ref[0] + d_ref[1], 1.0)
    mean = agg * (1.0 / deg)
    h = (jnp.dot(mean, wl_ref[...], preferred_element_type=jnp.float32)
         + bl_ref[...]
         + jnp.dot(x_ref[...], wr_ref[...], preferred_element_type=jnp.float32))
    h = jnp.maximum(h, 0.0)
    o_ref[...] = (jnp.dot(h, wo_ref[...], preferred_element_type=jnp.float32)
                  + bo_ref[...])


_BLK = 1000
_GRID = N_NODES // _BLK


def _tc_specs():
    a_spec = pl.BlockSpec((NC, _BLK, D), lambda i: (0, i, 0))
    d_spec = pl.BlockSpec((NC, _BLK, 1), lambda i: (0, i, 0))
    x_spec = pl.BlockSpec((_BLK, D), lambda i: (i, 0))
    w_spec = pl.BlockSpec((D, D), lambda i: (0, 0))
    b_spec = pl.BlockSpec((1, D), lambda i: (0, 0))
    o_spec = pl.BlockSpec((_BLK, D), lambda i: (i, 0))
    return a_spec, d_spec, x_spec, w_spec, b_spec, o_spec


def _tc_call_layer1(agg, deg, x, Wl, Wr, bl):
    a_spec, d_spec, x_spec, w_spec, b_spec, o_spec = _tc_specs()
    return pl.pallas_call(
        _tc_layer1,
        grid=(_GRID,),
        in_specs=[a_spec, d_spec, x_spec, w_spec, w_spec, b_spec],
        out_specs=o_spec,
        out_shape=jax.ShapeDtypeStruct((N_NODES, D), jnp.float32),
    )(agg, deg, x, Wl, Wr, bl)


def _tc_call_layer2(agg, deg, h, Wl, Wr, bl, Wlin, blin):
    a_spec, d_spec, x_spec, w_spec, b_spec, o_spec = _tc_specs()
    return pl.pallas_call(
        _tc_layer2,
        grid=(_GRID,),
        in_specs=[a_spec, d_spec, x_spec, w_spec, w_spec, b_spec, w_spec,
                  b_spec],
        out_specs=o_spec,
        out_shape=jax.ShapeDtypeStruct((N_NODES, D), jnp.float32),
    )(agg, deg, h, Wl, Wr, bl, Wlin, blin)


def kernel(x, edge_index, Wl1, bl1, Wr1, Wl2, bl2, Wr2, Wlin, blin):
    E = edge_index.shape[1]
    epad = NS * (NCH0 + NCH1) * CHUNK
    assert epad >= E

    src = edge_index[0].astype(jnp.int32)
    dst = edge_index[1].astype(jnp.int32)
    pad = epad - E
    if pad:
        src = jnp.concatenate([src, jnp.zeros((pad,), jnp.int32)])
        dst = jnp.concatenate([dst, jnp.full((pad,), DUMMY_ROW, jnp.int32)])

    zrows = jnp.zeros((ROWS_PER_TILE, D), jnp.float32)

    agg1, deg1 = _make_sc_agg(True)(x, src, dst, zrows)
    deg3 = deg1.reshape(NC, NR, 1)
    h = _tc_call_layer1(agg1, deg3, x, Wl1, Wr1, bl1.reshape(1, D))

    agg2 = _make_sc_agg(False)(h, src, dst, zrows)
    out = _tc_call_layer2(agg2, deg3, h, Wl2, Wr2, bl2.reshape(1, D),
                          Wlin, blin.reshape(1, D))
    return out


# speedup vs baseline: 1.3509x; 1.3509x over previous
import functools

import jax
import jax.numpy as jnp
from jax import lax
from jax.experimental import pallas as pl
from jax.experimental.pallas import tpu as pltpu
from jax.experimental.pallas import tpu_sc as plsc

N_NODES = 10000
D = 128
NC = 2
NS = 16
NW = NC * NS
CHUNK = 120
NBUF = 3
NIDX = 6
NCH0 = 114
NCH1 = 54
ROWS_PER_TILE = 632
NR = NS * ROWS_PER_TILE
DUMMY_ROW = N_NODES


def _sc_agg_body(with_deg, feat, src, dst, zrows, *refs):
    if with_deg:
        (agg_out, deg_out, idx_s, idx_d, ones_v, zdeg_v, agg_sp,
         deg_sp) = refs[:8]
        rest = refs[8:]
    else:
        (agg_out, idx_s, idx_d, agg_sp) = refs[:4]
        rest = refs[4:]
    rows = rest[:NBUF]
    gsems = rest[NBUF:2 * NBUF]
    isems = rest[2 * NBUF:2 * NBUF + NIDX]

    cid = lax.axis_index("c")
    sid = lax.axis_index("s")
    n_chunks = jnp.where(cid == 0, NCH0, NCH1)
    row0 = jnp.where(cid == 0, sid * NCH0, NS * NCH0 + sid * NCH1)

    def fetch_idx(slot, k, sem):
        base = (row0 + k) * CHUNK
        pltpu.async_copy(src.at[pl.ds(base, CHUNK)], idx_s.at[slot], sem)
        pltpu.async_copy(dst.at[pl.ds(base, CHUNK)], idx_d.at[slot], sem)

    def wait_idx(slot, sem):
        pltpu.make_async_copy(src.at[pl.ds(0, CHUNK)], idx_s.at[slot],
                              sem).wait()
        pltpu.make_async_copy(dst.at[pl.ds(0, CHUNK)], idx_d.at[slot],
                              sem).wait()

    for r in range(NIDX):
        @pl.when(r < n_chunks)
        def _prime_fetch(r=r):
            fetch_idx(r, r, isems[r])

    base_r = sid * ROWS_PER_TILE
    pltpu.sync_copy(zrows, agg_sp.at[pl.ds(base_r, ROWS_PER_TILE)])
    if with_deg:
        for j in range(8):
            ones_v[pl.ds(j * 16, 16)] = jnp.ones((16,), jnp.float32)
        for j in range(640 // 16):
            zdeg_v[pl.ds(j * 16, 16)] = jnp.zeros((16,), jnp.float32)
        pltpu.sync_copy(zdeg_v.at[pl.ds(0, ROWS_PER_TILE)],
                        deg_sp.at[pl.ds(base_r, ROWS_PER_TILE)])
    plsc.subcore_barrier()

    for b in range(NBUF):
        @pl.when(b < n_chunks)
        def _prime_gather(b=b):
            wait_idx(b, isems[b])
            pltpu.async_copy(feat.at[idx_s.at[b]], rows[b], gsems[b])

    def round_(g, carry):
        for b in range(NIDX):
            k = g + b

            @pl.when(k < n_chunks)
            def _consume(b=b, k=k, rb=b % NBUF):
                pltpu.make_async_copy(feat.at[idx_s.at[b]], rows[rb],
                                      gsems[rb]).wait()
                pltpu.sync_copy(rows[rb], agg_sp.at[idx_d.at[b]], add=True)
                if with_deg:
                    pltpu.sync_copy(ones_v.at[pl.ds(0, CHUNK)],
                                    deg_sp.at[idx_d.at[b]], add=True)

            @pl.when(k + NIDX < n_chunks)
            def _refetch(b=b, k=k):
                fetch_idx(b, k + NIDX, isems[b])

            @pl.when(k + NBUF < n_chunks)
            def _issue_next(b=b, k=k, rb=b % NBUF):
                nslot = (b + NBUF) % NIDX
                wait_idx(nslot, isems[nslot])
                pltpu.async_copy(feat.at[idx_s.at[nslot]], rows[rb],
                                 gsems[rb])
        return carry

    max_nch = max(NCH0, NCH1)
    lax.fori_loop(0, max_nch // NIDX, lambda i, c: round_(i * NIDX, c), 0)
    plsc.subcore_barrier()

    pltpu.sync_copy(agg_sp.at[pl.ds(base_r, ROWS_PER_TILE)],
                    agg_out.at[cid, pl.ds(base_r, ROWS_PER_TILE)])
    if with_deg:
        pltpu.sync_copy(deg_sp.at[pl.ds(base_r, ROWS_PER_TILE)],
                        zdeg_v.at[pl.ds(0, ROWS_PER_TILE)])
        pltpu.sync_copy(zdeg_v.at[pl.ds(0, ROWS_PER_TILE)],
                        deg_out.at[pl.ds(cid * NR + base_r, ROWS_PER_TILE)])


def _make_sc_agg(with_deg):
    mesh = plsc.VectorSubcoreMesh(core_axis_name="c", subcore_axis_name="s")
    out_type = [jax.ShapeDtypeStruct((NC, NR, D), jnp.float32)]
    scratch = [
        pltpu.VMEM((NIDX, CHUNK), jnp.int32),
        pltpu.VMEM((NIDX, CHUNK), jnp.int32),
    ]
    if with_deg:
        out_type.append(jax.ShapeDtypeStruct((NC * NR,), jnp.float32))
        scratch.append(pltpu.VMEM((128,), jnp.float32))
        scratch.append(pltpu.VMEM((640,), jnp.float32))
        scratch.append(pltpu.VMEM_SHARED((NR, D), jnp.float32))
        scratch.append(pltpu.VMEM_SHARED((NR,), jnp.float32))
    else:
        scratch.append(pltpu.VMEM_SHARED((NR, D), jnp.float32))
    scratch += [pltpu.VMEM((CHUNK, D), jnp.float32) for _ in range(NBUF)]
    scratch += [pltpu.SemaphoreType.DMA for _ in range(NBUF + NIDX)]
    return pl.kernel(
        functools.partial(_sc_agg_body, with_deg),
        out_type=out_type if with_deg else out_type[0],
        mesh=mesh,
        scratch_types=scratch,
    )


def _tc_layer1(a_ref, d_ref, x_ref, wl_ref, wr_ref, bl_ref, o_ref):
    agg = a_ref[0] + a_ref[1]
    deg = jnp.maximum(d_ref[0] + d_ref[1], 1.0)
    mean = agg * (1.0 / deg)
    h = (jnp.dot(mean, wl_ref[...], preferred_element_type=jnp.float32)
         + bl_ref[...]
         + jnp.dot(x_ref[...], wr_ref[...], preferred_element_type=jnp.float32))
    o_ref[...] = jnp.maximum(h, 0.0)


def _tc_layer2(a_ref, d_ref, x_ref, wl_ref, wr_ref, bl_ref, wo_ref, bo_ref,
               o_ref):
    agg = a_ref[0] + a_ref[1]
    deg = jnp.maximum(d_ref[0] + d_ref[1], 1.0)
    mean = agg * (1.0 / deg)
    h = (jnp.dot(mean, wl_ref[...], preferred_element_type=jnp.float32)
         + bl_ref[...]
         + jnp.dot(x_ref[...], wr_ref[...], preferred_element_type=jnp.float32))
    h = jnp.maximum(h, 0.0)
    o_ref[...] = (jnp.dot(h, wo_ref[...], preferred_element_type=jnp.float32)
                  + bo_ref[...])


_BLK = 1000
_GRID = N_NODES // _BLK


def _tc_specs():
    a_spec = pl.BlockSpec((NC, _BLK, D), lambda i: (0, i, 0))
    d_spec = pl.BlockSpec((NC, _BLK, 1), lambda i: (0, i, 0))
    x_spec = pl.BlockSpec((_BLK, D), lambda i: (i, 0))
    w_spec = pl.BlockSpec((D, D), lambda i: (0, 0))
    b_spec = pl.BlockSpec((1, D), lambda i: (0, 0))
    o_spec = pl.BlockSpec((_BLK, D), lambda i: (i, 0))
    return a_spec, d_spec, x_spec, w_spec, b_spec, o_spec


def _tc_call_layer1(agg, deg, x, Wl, Wr, bl):
    a_spec, d_spec, x_spec, w_spec, b_spec, o_spec = _tc_specs()
    return pl.pallas_call(
        _tc_layer1,
        grid=(_GRID,),
        in_specs=[a_spec, d_spec, x_spec, w_spec, w_spec, b_spec],
        out_specs=o_spec,
        out_shape=jax.ShapeDtypeStruct((N_NODES, D), jnp.float32),
    )(agg, deg, x, Wl, Wr, bl)


def _tc_call_layer2(agg, deg, h, Wl, Wr, bl, Wlin, blin):
    a_spec, d_spec, x_spec, w_spec, b_spec, o_spec = _tc_specs()
    return pl.pallas_call(
        _tc_layer2,
        grid=(_GRID,),
        in_specs=[a_spec, d_spec, x_spec, w_spec, w_spec, b_spec, w_spec,
                  b_spec],
        out_specs=o_spec,
        out_shape=jax.ShapeDtypeStruct((N_NODES, D), jnp.float32),
    )(agg, deg, h, Wl, Wr, bl, Wlin, blin)


def kernel(x, edge_index, Wl1, bl1, Wr1, Wl2, bl2, Wr2, Wlin, blin):
    E = edge_index.shape[1]
    epad = NS * (NCH0 + NCH1) * CHUNK
    assert epad >= E

    src = edge_index[0].astype(jnp.int32)
    dst = edge_index[1].astype(jnp.int32)
    pad = epad - E
    if pad:
        src = jnp.concatenate([src, jnp.zeros((pad,), jnp.int32)])
        dst = jnp.concatenate([dst, jnp.full((pad,), DUMMY_ROW, jnp.int32)])

    zrows = jnp.zeros((ROWS_PER_TILE, D), jnp.float32)

    agg1, deg1 = _make_sc_agg(True)(x, src, dst, zrows)
    deg3 = deg1.reshape(NC, NR, 1)
    h = _tc_call_layer1(agg1, deg3, x, Wl1, Wr1, bl1.reshape(1, D))

    agg2 = _make_sc_agg(False)(h, src, dst, zrows)
    out = _tc_call_layer2(agg2, deg3, h, Wl2, Wr2, bl2.reshape(1, D),
                          Wlin, blin.reshape(1, D))
    return out


# speedup vs baseline: 1.3921x; 1.0306x over previous
import functools

import jax
import jax.numpy as jnp
from jax import lax
from jax.experimental import pallas as pl
from jax.experimental.pallas import tpu as pltpu
from jax.experimental.pallas import tpu_sc as plsc

N_NODES = 10000
D = 128
NC = 2
NS = 16
NW = NC * NS
CHUNK = 120
NBUF = 3
NIDX = 6
NCH0 = 126
NCH1 = 42
ROWS_PER_TILE = 632
NR = NS * ROWS_PER_TILE
DUMMY_ROW = N_NODES


def _sc_agg_body(with_deg, feat, src, dst, zrows, *refs):
    if with_deg:
        (agg_out, deg_out, idx_s, idx_d, ones_v, zdeg_v, agg_sp,
         deg_sp) = refs[:8]
        rest = refs[8:]
    else:
        (agg_out, idx_s, idx_d, agg_sp) = refs[:4]
        rest = refs[4:]
    rows = rest[:NBUF]
    gsems = rest[NBUF:2 * NBUF]
    isems = rest[2 * NBUF:2 * NBUF + NIDX]

    cid = lax.axis_index("c")
    sid = lax.axis_index("s")
    n_chunks = jnp.where(cid == 0, NCH0, NCH1)
    row0 = jnp.where(cid == 0, sid * NCH0, NS * NCH0 + sid * NCH1)

    def fetch_idx(slot, k, sem):
        base = (row0 + k) * CHUNK
        pltpu.async_copy(src.at[pl.ds(base, CHUNK)], idx_s.at[slot], sem)
        pltpu.async_copy(dst.at[pl.ds(base, CHUNK)], idx_d.at[slot], sem)

    def wait_idx(slot, sem):
        pltpu.make_async_copy(src.at[pl.ds(0, CHUNK)], idx_s.at[slot],
                              sem).wait()
        pltpu.make_async_copy(dst.at[pl.ds(0, CHUNK)], idx_d.at[slot],
                              sem).wait()

    for r in range(NIDX):
        @pl.when(r < n_chunks)
        def _prime_fetch(r=r):
            fetch_idx(r, r, isems[r])

    base_r = sid * ROWS_PER_TILE
    pltpu.sync_copy(zrows, agg_sp.at[pl.ds(base_r, ROWS_PER_TILE)])
    if with_deg:
        for j in range(8):
            ones_v[pl.ds(j * 16, 16)] = jnp.ones((16,), jnp.float32)
        for j in range(640 // 16):
            zdeg_v[pl.ds(j * 16, 16)] = jnp.zeros((16,), jnp.float32)
        pltpu.sync_copy(zdeg_v.at[pl.ds(0, ROWS_PER_TILE)],
                        deg_sp.at[pl.ds(base_r, ROWS_PER_TILE)])
    plsc.subcore_barrier()

    for b in range(NBUF):
        @pl.when(b < n_chunks)
        def _prime_gather(b=b):
            wait_idx(b, isems[b])
            pltpu.async_copy(feat.at[idx_s.at[b]], rows[b], gsems[b])

    def round_(g, carry):
        for b in range(NIDX):
            k = g + b

            @pl.when(k < n_chunks)
            def _consume(b=b, k=k, rb=b % NBUF):
                pltpu.make_async_copy(feat.at[idx_s.at[b]], rows[rb],
                                      gsems[rb]).wait()
                pltpu.sync_copy(rows[rb], agg_sp.at[idx_d.at[b]], add=True)
                if with_deg:
                    pltpu.sync_copy(ones_v.at[pl.ds(0, CHUNK)],
                                    deg_sp.at[idx_d.at[b]], add=True)

            @pl.when(k + NIDX < n_chunks)
            def _refetch(b=b, k=k):
                fetch_idx(b, k + NIDX, isems[b])

            @pl.when(k + NBUF < n_chunks)
            def _issue_next(b=b, k=k, rb=b % NBUF):
                nslot = (b + NBUF) % NIDX
                wait_idx(nslot, isems[nslot])
                pltpu.async_copy(feat.at[idx_s.at[nslot]], rows[rb],
                                 gsems[rb])
        return carry

    max_nch = max(NCH0, NCH1)
    lax.fori_loop(0, max_nch // NIDX, lambda i, c: round_(i * NIDX, c), 0)
    plsc.subcore_barrier()

    pltpu.sync_copy(agg_sp.at[pl.ds(base_r, ROWS_PER_TILE)],
                    agg_out.at[cid, pl.ds(base_r, ROWS_PER_TILE)])
    if with_deg:
        pltpu.sync_copy(deg_sp.at[pl.ds(base_r, ROWS_PER_TILE)],
                        zdeg_v.at[pl.ds(0, ROWS_PER_TILE)])
        pltpu.sync_copy(zdeg_v.at[pl.ds(0, ROWS_PER_TILE)],
                        deg_out.at[pl.ds(cid * NR + base_r, ROWS_PER_TILE)])


def _make_sc_agg(with_deg):
    mesh = plsc.VectorSubcoreMesh(core_axis_name="c", subcore_axis_name="s")
    out_type = [jax.ShapeDtypeStruct((NC, NR, D), jnp.float32)]
    scratch = [
        pltpu.VMEM((NIDX, CHUNK), jnp.int32),
        pltpu.VMEM((NIDX, CHUNK), jnp.int32),
    ]
    if with_deg:
        out_type.append(jax.ShapeDtypeStruct((NC * NR,), jnp.float32))
        scratch.append(pltpu.VMEM((128,), jnp.float32))
        scratch.append(pltpu.VMEM((640,), jnp.float32))
        scratch.append(pltpu.VMEM_SHARED((NR, D), jnp.float32))
        scratch.append(pltpu.VMEM_SHARED((NR,), jnp.float32))
    else:
        scratch.append(pltpu.VMEM_SHARED((NR, D), jnp.float32))
    scratch += [pltpu.VMEM((CHUNK, D), jnp.float32) for _ in range(NBUF)]
    scratch += [pltpu.SemaphoreType.DMA for _ in range(NBUF + NIDX)]
    return pl.kernel(
        functools.partial(_sc_agg_body, with_deg),
        out_type=out_type if with_deg else out_type[0],
        mesh=mesh,
        scratch_types=scratch,
    )


def _tc_layer1(a_ref, d_ref, x_ref, wl_ref, wr_ref, bl_ref, o_ref):
    agg = a_ref[0] + a_ref[1]
    deg = jnp.maximum(d_ref[0] + d_ref[1], 1.0)
    mean = agg * (1.0 / deg)
    h = (jnp.dot(mean, wl_ref[...], preferred_element_type=jnp.float32)
         + bl_ref[...]
         + jnp.dot(x_ref[...], wr_ref[...], preferred_element_type=jnp.float32))
    o_ref[...] = jnp.maximum(h, 0.0)


def _tc_layer2(a_ref, d_ref, x_ref, wl_ref, wr_ref, bl_ref, wo_ref, bo_ref,
               o_ref):
    agg = a_ref[0] + a_ref[1]
    deg = jnp.maximum(d_ref[0] + d_ref[1], 1.0)
    mean = agg * (1.0 / deg)
    h = (jnp.dot(mean, wl_ref[...], preferred_element_type=jnp.float32)
         + bl_ref[...]
         + jnp.dot(x_ref[...], wr_ref[...], preferred_element_type=jnp.float32))
    h = jnp.maximum(h, 0.0)
    o_ref[...] = (jnp.dot(h, wo_ref[...], preferred_element_type=jnp.float32)
                  + bo_ref[...])


_BLK = 1000
_GRID = N_NODES // _BLK


def _tc_specs():
    a_spec = pl.BlockSpec((NC, _BLK, D), lambda i: (0, i, 0))
    d_spec = pl.BlockSpec((NC, _BLK, 1), lambda i: (0, i, 0))
    x_spec = pl.BlockSpec((_BLK, D), lambda i: (i, 0))
    w_spec = pl.BlockSpec((D, D), lambda i: (0, 0))
    b_spec = pl.BlockSpec((1, D), lambda i: (0, 0))
    o_spec = pl.BlockSpec((_BLK, D), lambda i: (i, 0))
    return a_spec, d_spec, x_spec, w_spec, b_spec, o_spec


def _tc_call_layer1(agg, deg, x, Wl, Wr, bl):
    a_spec, d_spec, x_spec, w_spec, b_spec, o_spec = _tc_specs()
    return pl.pallas_call(
        _tc_layer1,
        grid=(_GRID,),
        in_specs=[a_spec, d_spec, x_spec, w_spec, w_spec, b_spec],
        out_specs=o_spec,
        out_shape=jax.ShapeDtypeStruct((N_NODES, D), jnp.float32),
    )(agg, deg, x, Wl, Wr, bl)


def _tc_call_layer2(agg, deg, h, Wl, Wr, bl, Wlin, blin):
    a_spec, d_spec, x_spec, w_spec, b_spec, o_spec = _tc_specs()
    return pl.pallas_call(
        _tc_layer2,
        grid=(_GRID,),
        in_specs=[a_spec, d_spec, x_spec, w_spec, w_spec, b_spec, w_spec,
                  b_spec],
        out_specs=o_spec,
        out_shape=jax.ShapeDtypeStruct((N_NODES, D), jnp.float32),
    )(agg, deg, h, Wl, Wr, bl, Wlin, blin)


def kernel(x, edge_index, Wl1, bl1, Wr1, Wl2, bl2, Wr2, Wlin, blin):
    E = edge_index.shape[1]
    epad = NS * (NCH0 + NCH1) * CHUNK
    assert epad >= E

    src = edge_index[0].astype(jnp.int32)
    dst = edge_index[1].astype(jnp.int32)
    pad = epad - E
    if pad:
        src = jnp.concatenate([src, jnp.zeros((pad,), jnp.int32)])
        dst = jnp.concatenate([dst, jnp.full((pad,), DUMMY_ROW, jnp.int32)])

    zrows = jnp.zeros((ROWS_PER_TILE, D), jnp.float32)

    agg1, deg1 = _make_sc_agg(True)(x, src, dst, zrows)
    deg3 = deg1.reshape(NC, NR, 1)
    h = _tc_call_layer1(agg1, deg3, x, Wl1, Wr1, bl1.reshape(1, D))

    agg2 = _make_sc_agg(False)(h, src, dst, zrows)
    out = _tc_call_layer2(agg2, deg3, h, Wl2, Wr2, bl2.reshape(1, D),
                          Wlin, blin.reshape(1, D))
    return out


# speedup vs baseline: 1.4257x; 1.0241x over previous
import functools

import jax
import jax.numpy as jnp
from jax import lax
from jax.experimental import pallas as pl
from jax.experimental.pallas import tpu as pltpu
from jax.experimental.pallas import tpu_sc as plsc

N_NODES = 10000
D = 128
NC = 2
NS = 16
NW = NC * NS
CHUNK = 120
NBUF = 3
NIDX = 6
NCH0 = 138
NCH1 = 30
ROWS_PER_TILE = 632
NR = NS * ROWS_PER_TILE
DUMMY_ROW = N_NODES


def _sc_agg_body(with_deg, feat, src, dst, zrows, *refs):
    if with_deg:
        (agg_out, deg_out, idx_s, idx_d, ones_v, zdeg_v, agg_sp,
         deg_sp) = refs[:8]
        rest = refs[8:]
    else:
        (agg_out, idx_s, idx_d, agg_sp) = refs[:4]
        rest = refs[4:]
    rows = rest[:NBUF]
    gsems = rest[NBUF:2 * NBUF]
    isems = rest[2 * NBUF:2 * NBUF + NIDX]

    cid = lax.axis_index("c")
    sid = lax.axis_index("s")
    n_chunks = jnp.where(cid == 0, NCH0, NCH1)
    row0 = jnp.where(cid == 0, sid * NCH0, NS * NCH0 + sid * NCH1)

    def fetch_idx(slot, k, sem):
        base = (row0 + k) * CHUNK
        pltpu.async_copy(src.at[pl.ds(base, CHUNK)], idx_s.at[slot], sem)
        pltpu.async_copy(dst.at[pl.ds(base, CHUNK)], idx_d.at[slot], sem)

    def wait_idx(slot, sem):
        pltpu.make_async_copy(src.at[pl.ds(0, CHUNK)], idx_s.at[slot],
                              sem).wait()
        pltpu.make_async_copy(dst.at[pl.ds(0, CHUNK)], idx_d.at[slot],
                              sem).wait()

    for r in range(NIDX):
        @pl.when(r < n_chunks)
        def _prime_fetch(r=r):
            fetch_idx(r, r, isems[r])

    base_r = sid * ROWS_PER_TILE
    pltpu.sync_copy(zrows, agg_sp.at[pl.ds(base_r, ROWS_PER_TILE)])
    if with_deg:
        for j in range(8):
            ones_v[pl.ds(j * 16, 16)] = jnp.ones((16,), jnp.float32)
        for j in range(640 // 16):
            zdeg_v[pl.ds(j * 16, 16)] = jnp.zeros((16,), jnp.float32)
        pltpu.sync_copy(zdeg_v.at[pl.ds(0, ROWS_PER_TILE)],
                        deg_sp.at[pl.ds(base_r, ROWS_PER_TILE)])
    plsc.subcore_barrier()

    for b in range(NBUF):
        @pl.when(b < n_chunks)
        def _prime_gather(b=b):
            wait_idx(b, isems[b])
            pltpu.async_copy(feat.at[idx_s.at[b]], rows[b], gsems[b])

    def round_(g, carry):
        for b in range(NIDX):
            k = g + b

            @pl.when(k < n_chunks)
            def _consume(b=b, k=k, rb=b % NBUF):
                pltpu.make_async_copy(feat.at[idx_s.at[b]], rows[rb],
                                      gsems[rb]).wait()
                pltpu.sync_copy(rows[rb], agg_sp.at[idx_d.at[b]], add=True)
                if with_deg:
                    pltpu.sync_copy(ones_v.at[pl.ds(0, CHUNK)],
                                    deg_sp.at[idx_d.at[b]], add=True)

            @pl.when(k + NIDX < n_chunks)
            def _refetch(b=b, k=k):
                fetch_idx(b, k + NIDX, isems[b])

            @pl.when(k + NBUF < n_chunks)
            def _issue_next(b=b, k=k, rb=b % NBUF):
                nslot = (b + NBUF) % NIDX
                wait_idx(nslot, isems[nslot])
                pltpu.async_copy(feat.at[idx_s.at[nslot]], rows[rb],
                                 gsems[rb])
        return carry

    max_nch = max(NCH0, NCH1)
    lax.fori_loop(0, max_nch // NIDX, lambda i, c: round_(i * NIDX, c), 0)
    plsc.subcore_barrier()

    pltpu.sync_copy(agg_sp.at[pl.ds(base_r, ROWS_PER_TILE)],
                    agg_out.at[cid, pl.ds(base_r, ROWS_PER_TILE)])
    if with_deg:
        pltpu.sync_copy(deg_sp.at[pl.ds(base_r, ROWS_PER_TILE)],
                        zdeg_v.at[pl.ds(0, ROWS_PER_TILE)])
        pltpu.sync_copy(zdeg_v.at[pl.ds(0, ROWS_PER_TILE)],
                        deg_out.at[pl.ds(cid * NR + base_r, ROWS_PER_TILE)])


def _make_sc_agg(with_deg):
    mesh = plsc.VectorSubcoreMesh(core_axis_name="c", subcore_axis_name="s")
    out_type = [jax.ShapeDtypeStruct((NC, NR, D), jnp.float32)]
    scratch = [
        pltpu.VMEM((NIDX, CHUNK), jnp.int32),
        pltpu.VMEM((NIDX, CHUNK), jnp.int32),
    ]
    if with_deg:
        out_type.append(jax.ShapeDtypeStruct((NC * NR,), jnp.float32))
        scratch.append(pltpu.VMEM((128,), jnp.float32))
        scratch.append(pltpu.VMEM((640,), jnp.float32))
        scratch.append(pltpu.VMEM_SHARED((NR, D), jnp.float32))
        scratch.append(pltpu.VMEM_SHARED((NR,), jnp.float32))
    else:
        scratch.append(pltpu.VMEM_SHARED((NR, D), jnp.float32))
    scratch += [pltpu.VMEM((CHUNK, D), jnp.float32) for _ in range(NBUF)]
    scratch += [pltpu.SemaphoreType.DMA for _ in range(NBUF + NIDX)]
    return pl.kernel(
        functools.partial(_sc_agg_body, with_deg),
        out_type=out_type if with_deg else out_type[0],
        mesh=mesh,
        scratch_types=scratch,
    )


def _tc_layer1(a_ref, d_ref, x_ref, wl_ref, wr_ref, bl_ref, o_ref):
    agg = a_ref[0] + a_ref[1]
    deg = jnp.maximum(d_ref[0] + d_ref[1], 1.0)
    mean = agg * (1.0 / deg)
    h = (jnp.dot(mean, wl_ref[...], preferred_element_type=jnp.float32)
         + bl_ref[...]
         + jnp.dot(x_ref[...], wr_ref[...], preferred_element_type=jnp.float32))
    o_ref[...] = jnp.maximum(h, 0.0)


def _tc_layer2(a_ref, d_ref, x_ref, wl_ref, wr_ref, bl_ref, wo_ref, bo_ref,
               o_ref):
    agg = a_ref[0] + a_ref[1]
    deg = jnp.maximum(d_ref[0] + d_ref[1], 1.0)
    mean = agg * (1.0 / deg)
    h = (jnp.dot(mean, wl_ref[...], preferred_element_type=jnp.float32)
         + bl_ref[...]
         + jnp.dot(x_ref[...], wr_ref[...], preferred_element_type=jnp.float32))
    h = jnp.maximum(h, 0.0)
    o_ref[...] = (jnp.dot(h, wo_ref[...], preferred_element_type=jnp.float32)
                  + bo_ref[...])


_BLK = 1000
_GRID = N_NODES // _BLK


def _tc_specs():
    a_spec = pl.BlockSpec((NC, _BLK, D), lambda i: (0, i, 0))
    d_spec = pl.BlockSpec((NC, _BLK, 1), lambda i: (0, i, 0))
    x_spec = pl.BlockSpec((_BLK, D), lambda i: (i, 0))
    w_spec = pl.BlockSpec((D, D), lambda i: (0, 0))
    b_spec = pl.BlockSpec((1, D), lambda i: (0, 0))
    o_spec = pl.BlockSpec((_BLK, D), lambda i: (i, 0))
    return a_spec, d_spec, x_spec, w_spec, b_spec, o_spec


def _tc_call_layer1(agg, deg, x, Wl, Wr, bl):
    a_spec, d_spec, x_spec, w_spec, b_spec, o_spec = _tc_specs()
    return pl.pallas_call(
        _tc_layer1,
        grid=(_GRID,),
        in_specs=[a_spec, d_spec, x_spec, w_spec, w_spec, b_spec],
        out_specs=o_spec,
        out_shape=jax.ShapeDtypeStruct((N_NODES, D), jnp.float32),
    )(agg, deg, x, Wl, Wr, bl)


def _tc_call_layer2(agg, deg, h, Wl, Wr, bl, Wlin, blin):
    a_spec, d_spec, x_spec, w_spec, b_spec, o_spec = _tc_specs()
    return pl.pallas_call(
        _tc_layer2,
        grid=(_GRID,),
        in_specs=[a_spec, d_spec, x_spec, w_spec, w_spec, b_spec, w_spec,
                  b_spec],
        out_specs=o_spec,
        out_shape=jax.ShapeDtypeStruct((N_NODES, D), jnp.float32),
    )(agg, deg, h, Wl, Wr, bl, Wlin, blin)


def kernel(x, edge_index, Wl1, bl1, Wr1, Wl2, bl2, Wr2, Wlin, blin):
    E = edge_index.shape[1]
    epad = NS * (NCH0 + NCH1) * CHUNK
    assert epad >= E

    src = edge_index[0].astype(jnp.int32)
    dst = edge_index[1].astype(jnp.int32)
    pad = epad - E
    if pad:
        src = jnp.concatenate([src, jnp.zeros((pad,), jnp.int32)])
        dst = jnp.concatenate([dst, jnp.full((pad,), DUMMY_ROW, jnp.int32)])

    zrows = jnp.zeros((ROWS_PER_TILE, D), jnp.float32)

    agg1, deg1 = _make_sc_agg(True)(x, src, dst, zrows)
    deg3 = deg1.reshape(NC, NR, 1)
    h = _tc_call_layer1(agg1, deg3, x, Wl1, Wr1, bl1.reshape(1, D))

    agg2 = _make_sc_agg(False)(h, src, dst, zrows)
    out = _tc_call_layer2(agg2, deg3, h, Wl2, Wr2, bl2.reshape(1, D),
                          Wlin, blin.reshape(1, D))
    return out
